# baseline (speedup 1.0000x reference)
import functools

import jax
import jax.numpy as jnp
from jax import lax
from jax.experimental import pallas as pl
from jax.experimental.pallas import tpu as pltpu
from jax.experimental.pallas import tpu_sc as plsc

_NUM_CORES = 2
_NUM_SUBCORES = 16
_NUM_WORKERS = _NUM_CORES * _NUM_SUBCORES

_BC = 32768


def _tc_transpose_pairs(entT):
    F, E = entT.shape
    grid = (E + _BC - 1) // _BC
    half = _BC // 2
    quart = _BC // 4

    def bf16_bits(v):
        u = jax.lax.bitcast_convert_type(v, jnp.uint32)
        return (u + 0x7FFF + ((u >> 16) & 1)) >> 16

    def body(x_ref, o_ref):
        z = jnp.concatenate([x_ref[:, :half], x_ref[:, half:]], axis=0)
        y = jnp.transpose(z)
        packed = (bf16_bits(y[quart:, :]) << 16) | bf16_bits(y[:quart, :])
        o_ref[...] = jax.lax.bitcast_convert_type(packed, jnp.int32)

    return pl.pallas_call(
        body,
        grid=(grid,),
        in_specs=[pl.BlockSpec((F, _BC), lambda j: (0, j))],
        out_specs=pl.BlockSpec((quart, 2 * F), lambda j: (j, 0)),
        out_shape=jax.ShapeDtypeStruct((grid * quart, 2 * F), jnp.int32),
        compiler_params=pltpu.CompilerParams(
            dimension_semantics=("parallel",)),
    )(entT)


def _sc_gather(ent2, rn, hp, tp, r):
    B = hp.shape[0]
    W = rn.shape[1]
    bpw = B // _NUM_WORKERS
    C = 128
    n_chunks = bpw // C
    ent_t = jax.ShapeDtypeStruct((B, W), jnp.int32)
    rn_t = jax.ShapeDtypeStruct((B, W), jnp.float32)
    erow_t = pltpu.VMEM((C, W), jnp.int32)
    mesh = plsc.VectorSubcoreMesh(core_axis_name="c", subcore_axis_name="s")

    @functools.partial(
        pl.kernel,
        mesh=mesh,
        out_type=(ent_t, ent_t, rn_t),
        scratch_types=[
            pltpu.VMEM((bpw,), jnp.int32),
            pltpu.VMEM((bpw,), jnp.int32),
            pltpu.VMEM((bpw,), jnp.int32),
            ((erow_t, erow_t, pltpu.VMEM((C, W), jnp.float32)),) * 2,
            ((pltpu.SemaphoreType.DMA,) * 3,) * 2,
            ((pltpu.SemaphoreType.DMA,) * 3,) * 2,
        ],
    )
    def k(ent_hbm, rn_hbm, h_hbm, t_hbm, r_hbm,
          eh_o, et_o, rn_o, h_v, t_v, r_v, rows, gsem, wsem):
        wid = lax.axis_index("s") * _NUM_CORES + lax.axis_index("c")
        base = wid * bpw
        pltpu.sync_copy(h_hbm.at[pl.ds(base, bpw)], h_v)
        pltpu.sync_copy(t_hbm.at[pl.ds(base, bpw)], t_v)
        pltpu.sync_copy(r_hbm.at[pl.ds(base, bpw)], r_v)

        outs = (eh_o, et_o, rn_o)

        def do_chunk(c, b, drain):
            if drain:
                for i in range(3):
                    pltpu.make_async_copy(
                        rows[b][i], outs[i].at[pl.ds(base, C)],
                        wsem[b][i]).wait()
            csl = pl.ds(c * C, C)
            g0 = pltpu.async_copy(
                ent_hbm.at[h_v.at[csl]], rows[b][0], gsem[b][0])
            g1 = pltpu.async_copy(
                ent_hbm.at[t_v.at[csl]], rows[b][1], gsem[b][1])
            g2 = pltpu.async_copy(
                rn_hbm.at[r_v.at[csl]], rows[b][2], gsem[b][2])
            g0.wait()
            g1.wait()
            g2.wait()
            osl = pl.ds(base + c * C, C)
            for i in range(3):
                pltpu.async_copy(rows[b][i], outs[i].at[osl], wsem[b][i])

        do_chunk(0, 0, False)
        do_chunk(1, 1, False)

        @pl.loop(2, n_chunks, step=2)
        def _(c):
            do_chunk(c, 0, True)
            do_chunk(c + 1, 1, True)

        for b in range(2):
            for i in range(3):
                pltpu.make_async_copy(
                    rows[b][i], outs[i].at[pl.ds(base, C)],
                    wsem[b][i]).wait()

    return k(ent2, rn, hp, tp, r)


def _tc_math(eh2, et2, rn_g, sh, qh, st, qt, D):
    B = rn_g.shape[0]
    W = rn_g.shape[1]
    BT = 4096

    def body(eh_ref, et_ref, rn_ref, sh_ref, qh_ref, st_ref, qt_ref, o_ref):
        def pick(ref, s_ref, q_ref):
            packed = jax.lax.bitcast_convert_type(ref[...], jnp.uint32)
            lo = jax.lax.bitcast_convert_type(packed << 16, jnp.float32)
            hi = jax.lax.bitcast_convert_type(
                packed & jnp.uint32(0xFFFF0000), jnp.float32)
            row = jnp.where(s_ref[...] > 0, hi, lo)
            return jnp.where(q_ref[...] > 0, row[:, D:], row[:, :D])

        eh = pick(eh_ref, sh_ref, qh_ref)
        et = pick(et_ref, st_ref, qt_ref)
        rr = rn_ref[:, :D]
        nn = rn_ref[:, D:]
        dv = eh - et
        s = jnp.sum(dv * nn, axis=1, keepdims=True)
        o_ref[...] = jnp.abs(dv + rr - s * nn)

    ent_spec = pl.BlockSpec((BT, W), lambda i: (i, 0))
    row_spec = pl.BlockSpec((BT, W), lambda i: (i, 0))
    par_spec = pl.BlockSpec((BT, 1), lambda i: (i, 0))
    return pl.pallas_call(
        body,
        grid=(B // BT,),
        in_specs=[ent_spec] * 2 + [row_spec] + [par_spec] * 4,
        out_specs=pl.BlockSpec((BT, D), lambda i: (i, 0)),
        out_shape=jax.ShapeDtypeStruct((B, D), jnp.float32),
        compiler_params=pltpu.CompilerParams(
            dimension_semantics=("parallel",)),
    )(eh2, et2, rn_g, sh, qh, st, qt)


def kernel(h, t, r, ent_embeddings, rel_embeddings, normal_vectors):
    h = h.astype(jnp.int32)
    t = t.astype(jnp.int32)
    r = r.astype(jnp.int32)
    D = ent_embeddings.shape[1]
    ent2 = _tc_transpose_pairs(ent_embeddings.T)
    rn = jnp.concatenate([rel_embeddings, normal_vectors], axis=1)
    quart = _BC // 4
    hp = (h // _BC) * quart + (h % quart)
    tp = (t // _BC) * quart + (t % quart)
    eh2, et2, rn_g = _sc_gather(ent2, rn, hp, tp, r)
    sh = ((h // quart) & 1).reshape(-1, 1)
    st = ((t // quart) & 1).reshape(-1, 1)
    qh = ((h // (_BC // 2)) & 1).reshape(-1, 1)
    qt = ((t // (_BC // 2)) & 1).reshape(-1, 1)
    return _tc_math(eh2, et2, rn_g, sh, qh, st, qt, D)
